# baseline (speedup 1.0000x reference)
import functools

import jax
import jax.numpy as jnp
from jax import lax
from jax.experimental import pallas as pl
from jax.experimental.pallas import tpu as pltpu
from jax.experimental.pallas import tpu_sc as plsc

N = 10000
E = 320000
D = 128

NC = 2
NS = 16
NW = NC * NS
WIN = 128
NJ = 80
NG = NJ // 2
EPW = NJ * WIN
EPAD = NW * EPW
NTRASH = 10112
RPS = NTRASH // NS


def _mesh():
    return plsc.VectorSubcoreMesh(core_axis_name="c", subcore_axis_name="s")


_SC_PARAMS = pltpu.CompilerParams(needs_layout_passes=False)


@functools.partial(
    pl.kernel,
    mesh=_mesh(),
    compiler_params=_SC_PARAMS,
    out_type=jax.ShapeDtypeStruct((NC, NTRASH), jnp.float32),
    scratch_types=[
        pltpu.VMEM((NJ, WIN), jnp.int32),
        pltpu.VMEM((WIN,), jnp.float32),
        pltpu.VMEM((NTRASH,), jnp.float32),
        pltpu.VMEM_SHARED((NTRASH,), jnp.float32),
    ],
)
def _sc_deg(dst3, out, dst_buf, ones_v, zbuf, deg_sh):
    cid = lax.axis_index("c")
    sid = lax.axis_index("s")
    wid = cid * NS + sid

    def fill_ones(i, _):
        ones_v[pl.ds(i * 16, 16)] = jnp.ones((16,), jnp.float32)
        return 0

    lax.fori_loop(0, WIN // 16, fill_ones, 0)

    def fill_zero(i, _):
        zbuf[pl.ds(i * 16, 16)] = jnp.zeros((16,), jnp.float32)
        return 0

    lax.fori_loop(0, NTRASH // 16, fill_zero, 0)

    @pl.when(sid == 0)
    def _():
        pltpu.sync_copy(zbuf, deg_sh)

    plsc.subcore_barrier()
    pltpu.sync_copy(dst3.at[wid], dst_buf)

    def body(j, _):
        pltpu.sync_copy(ones_v, deg_sh.at[dst_buf.at[j]], add=True)
        return 0

    lax.fori_loop(0, NJ, body, 0)
    plsc.subcore_barrier()

    @pl.when(sid == 0)
    def _():
        pltpu.sync_copy(deg_sh, out.at[cid])


@functools.partial(
    pl.kernel,
    mesh=_mesh(),
    compiler_params=_SC_PARAMS,
    out_type=jax.ShapeDtypeStruct((NC, NTRASH, D), jnp.float32),
    scratch_types=[
        pltpu.VMEM((WIN,), jnp.int32),
        pltpu.VMEM((WIN,), jnp.int32),
        pltpu.VMEM((WIN,), jnp.int32),
        pltpu.VMEM((WIN,), jnp.int32),
        pltpu.VMEM((WIN, D), jnp.float32),
        pltpu.VMEM((WIN, D), jnp.float32),
        pltpu.SemaphoreType.DMA,
        pltpu.SemaphoreType.DMA,
        pltpu.SemaphoreType.DMA,
        pltpu.SemaphoreType.DMA,
        pltpu.VMEM_SHARED((NTRASH, D), jnp.float32),
    ],
)
def _sc_gcn(y_hbm, src3, dst3, out, src_w0, dst_w0, src_w1, dst_w1,
            rows0, rows1, gsem0, gsem1, ssem0, ssem1, acc_sh):
    cid = lax.axis_index("c")
    sid = lax.axis_index("s")
    wid = cid * NS + sid

    def fill_zero(i, _):
        for k in range(D // 16):
            rows0[i, pl.ds(k * 16, 16)] = jnp.zeros((16,), jnp.float32)
        return 0

    lax.fori_loop(0, WIN, fill_zero, 0)
    for k in range(RPS // WIN):
        pltpu.sync_copy(rows0, acc_sh.at[pl.ds(sid * RPS + k * WIN, WIN)])
    pltpu.sync_copy(rows0.at[pl.ds(0, RPS % WIN)],
                    acc_sh.at[pl.ds(sid * RPS + RPS - RPS % WIN, RPS % WIN)])
    plsc.subcore_barrier()

    pltpu.sync_copy(src3.at[wid, 0], src_w0)
    pltpu.sync_copy(dst3.at[wid, 0], dst_w0)
    g0 = pltpu.async_copy(y_hbm.at[src_w0], rows0, gsem0)

    def pair(g, _):
        j1 = 2 * g + 1
        pltpu.make_async_copy(y_hbm.at[src_w0], rows0, gsem0).wait()

        @pl.when(g > 0)
        def _():
            pltpu.make_async_copy(rows1, acc_sh.at[dst_w1], ssem1).wait()

        pltpu.sync_copy(src3.at[wid, j1], src_w1)
        pltpu.sync_copy(dst3.at[wid, j1], dst_w1)
        pltpu.async_copy(y_hbm.at[src_w1], rows1, gsem1)
        pltpu.async_copy(rows0, acc_sh.at[dst_w0], ssem0, add=True)
        pltpu.make_async_copy(rows0, acc_sh.at[dst_w0], ssem0).wait()

        @pl.when(g < NG - 1)
        def _():
            pltpu.sync_copy(src3.at[wid, 2 * g + 2], src_w0)
            pltpu.sync_copy(dst3.at[wid, 2 * g + 2], dst_w0)
            pltpu.async_copy(y_hbm.at[src_w0], rows0, gsem0)

        pltpu.make_async_copy(y_hbm.at[src_w1], rows1, gsem1).wait()
        pltpu.async_copy(rows1, acc_sh.at[dst_w1], ssem1, add=True)
        return 0

    lax.fori_loop(0, NG, pair, 0)
    pltpu.make_async_copy(rows1, acc_sh.at[dst_w1], ssem1).wait()
    plsc.subcore_barrier()
    pltpu.sync_copy(acc_sh.at[pl.ds(sid * RPS, RPS)],
                    out.at[cid, pl.ds(sid * RPS, RPS)])


@functools.partial(
    pl.kernel,
    mesh=_mesh(),
    compiler_params=_SC_PARAMS,
    out_type=[
        jax.ShapeDtypeStruct((NW, NJ, WIN), jnp.float32),
        jax.ShapeDtypeStruct((NC, NTRASH), jnp.float32),
    ],
    scratch_types=[
        pltpu.VMEM((NJ, WIN), jnp.int32),
        pltpu.VMEM((NJ, WIN), jnp.int32),
        pltpu.VMEM((NJ, WIN), jnp.float32),
        pltpu.VMEM((NTRASH,), jnp.float32),
        pltpu.VMEM((NTRASH,), jnp.float32),
        pltpu.VMEM((128,), jnp.float32),
        pltpu.VMEM_SHARED((NTRASH,), jnp.float32),
    ],
)
def _sc_gat0(src3, dst3, tabs_hbm, tabd_hbm, c_hbm, out_ee, out_den,
             src_buf, dst_buf, ee_buf, asrc_v, adst_v, c_v, den_sh):
    cid = lax.axis_index("c")
    sid = lax.axis_index("s")
    wid = cid * NS + sid

    def fill_zero(i, _):
        asrc_v[pl.ds(i * 16, 16)] = jnp.zeros((16,), jnp.float32)
        return 0

    lax.fori_loop(0, NTRASH // 16, fill_zero, 0)

    @pl.when(sid == 0)
    def _():
        pltpu.sync_copy(asrc_v, den_sh)

    plsc.subcore_barrier()

    pltpu.sync_copy(tabs_hbm, asrc_v)
    pltpu.sync_copy(tabd_hbm, adst_v)
    pltpu.sync_copy(c_hbm, c_v)
    pltpu.sync_copy(src3.at[wid], src_buf)
    pltpu.sync_copy(dst3.at[wid], dst_buf)
    c16 = c_v[pl.ds(0, 16)]

    def body(j, _):
        for k in range(WIN // 16):
            sl = pl.ds(k * 16, 16)
            si = src_buf[j, sl]
            di = dst_buf[j, sl]
            t = plsc.load_gather(asrc_v, [si]) + plsc.load_gather(adst_v, [di])
            t = jnp.where(t > 0, t, 0.2 * t)
            ee_buf[j, sl] = jnp.exp(t - c16)
        pltpu.sync_copy(ee_buf.at[j], den_sh.at[dst_buf.at[j]], add=True)
        return 0

    lax.fori_loop(0, NJ, body, 0)
    pltpu.sync_copy(ee_buf, out_ee.at[wid])
    plsc.subcore_barrier()

    @pl.when(sid == 0)
    def _():
        pltpu.sync_copy(den_sh, out_den.at[cid])


@functools.partial(
    pl.kernel,
    mesh=_mesh(),
    compiler_params=_SC_PARAMS,
    out_type=jax.ShapeDtypeStruct((NC, NTRASH, D), jnp.float32),
    scratch_types=[
        pltpu.VMEM((WIN,), jnp.int32),
        pltpu.VMEM((WIN,), jnp.int32),
        pltpu.VMEM((WIN,), jnp.int32),
        pltpu.VMEM((WIN,), jnp.int32),
        pltpu.VMEM((WIN, D), jnp.float32),
        pltpu.VMEM((WIN, D), jnp.float32),
        pltpu.VMEM((WIN,), jnp.float32),
        pltpu.VMEM((WIN,), jnp.float32),
        pltpu.SemaphoreType.DMA,
        pltpu.SemaphoreType.DMA,
        pltpu.SemaphoreType.DMA,
        pltpu.SemaphoreType.DMA,
        pltpu.VMEM_SHARED((NTRASH, D), jnp.float32),
    ],
)
def _sc_gat(zw_hbm, src3, dst3, ee3, out, src_w0, dst_w0, src_w1, dst_w1,
            rows0, rows1, ee_w0, ee_w1, gsem0, gsem1, ssem0, ssem1, acc_sh):
    cid = lax.axis_index("c")
    sid = lax.axis_index("s")
    wid = cid * NS + sid

    def fill_zero(i, _):
        for k in range(D // 16):
            rows0[i, pl.ds(k * 16, 16)] = jnp.zeros((16,), jnp.float32)
        return 0

    lax.fori_loop(0, WIN, fill_zero, 0)
    for k in range(RPS // WIN):
        pltpu.sync_copy(rows0, acc_sh.at[pl.ds(sid * RPS + k * WIN, WIN)])
    pltpu.sync_copy(rows0.at[pl.ds(0, RPS % WIN)],
                    acc_sh.at[pl.ds(sid * RPS + RPS - RPS % WIN, RPS % WIN)])
    plsc.subcore_barrier()

    def scale(rows, ee_w):
        def scale_grp(gi, _):
            ee16 = ee_w[pl.ds(gi * 16, 16)]
            for u in range(16):
                rr = gi * 16 + u
                s = jnp.take_along_axis(
                    ee16, jnp.full((16,), u, jnp.int32), axis=0)
                for k in range(D // 16):
                    sl = pl.ds(k * 16, 16)
                    rows[rr, sl] = rows[rr, sl] * s
            return 0

        lax.fori_loop(0, WIN // 16, scale_grp, 0)

    pltpu.sync_copy(src3.at[wid, 0], src_w0)
    pltpu.sync_copy(dst3.at[wid, 0], dst_w0)
    pltpu.sync_copy(ee3.at[wid, 0], ee_w0)
    pltpu.async_copy(zw_hbm.at[src_w0], rows0, gsem0)

    def pair(g, _):
        j0 = 2 * g
        j1 = j0 + 1
        pltpu.make_async_copy(zw_hbm.at[src_w0], rows0, gsem0).wait()
        scale(rows0, ee_w0)

        @pl.when(g > 0)
        def _():
            pltpu.make_async_copy(rows1, acc_sh.at[dst_w1], ssem1).wait()

        pltpu.sync_copy(src3.at[wid, j1], src_w1)
        pltpu.sync_copy(dst3.at[wid, j1], dst_w1)
        pltpu.sync_copy(ee3.at[wid, j1], ee_w1)
        pltpu.async_copy(zw_hbm.at[src_w1], rows1, gsem1)
        pltpu.async_copy(rows0, acc_sh.at[dst_w0], ssem0, add=True)
        pltpu.make_async_copy(zw_hbm.at[src_w1], rows1, gsem1).wait()
        scale(rows1, ee_w1)
        pltpu.make_async_copy(rows0, acc_sh.at[dst_w0], ssem0).wait()

        @pl.when(g < NG - 1)
        def _():
            pltpu.sync_copy(src3.at[wid, j0 + 2], src_w0)
            pltpu.sync_copy(dst3.at[wid, j0 + 2], dst_w0)
            pltpu.sync_copy(ee3.at[wid, j0 + 2], ee_w0)
            pltpu.async_copy(zw_hbm.at[src_w0], rows0, gsem0)

        pltpu.async_copy(rows1, acc_sh.at[dst_w1], ssem1, add=True)
        return 0

    lax.fori_loop(0, NG, pair, 0)
    pltpu.make_async_copy(rows1, acc_sh.at[dst_w1], ssem1).wait()
    plsc.subcore_barrier()
    pltpu.sync_copy(acc_sh.at[pl.ds(sid * RPS, RPS)],
                    out.at[cid, pl.ds(sid * RPS, RPS)])


def _tc1_body(x_ref, w_ref, degp_ref, y_ref):
    deg = degp_ref[0] + degp_ref[1] + 1.0
    dinv = 1.0 / jnp.sqrt(deg)
    xw = jnp.dot(x_ref[...], w_ref[...], preferred_element_type=jnp.float32)
    y_ref[...] = dinv[:, None] * xw


def _tc2_body(x_ref, y_ref, accp_ref, degp_ref, b1_ref, gnw_ref, gnb_ref,
              gnms_ref, gatw_ref, aw_src_ref, aw_dst_ref,
              x1_ref, zw_ref, as_ref, ad_ref, c_ref):
    deg = degp_ref[0] + degp_ref[1] + 1.0
    dinv = 1.0 / jnp.sqrt(deg)
    acc = accp_ref[0] + accp_ref[1]
    h = dinv[:, None] * (acc + y_ref[...]) + b1_ref[...]
    mean = jnp.mean(h, axis=0, keepdims=True)
    cen = h - mean * gnms_ref[...]
    var = jnp.mean(cen * cen, axis=0, keepdims=True)
    h = gnw_ref[...] * cen / jnp.sqrt(var + 1e-5) + gnb_ref[...]
    h = jnp.where(h > 0, h, 0.01 * h)
    x1 = x_ref[...] + h
    x1_ref[...] = x1
    zw = jnp.dot(x1, gatw_ref[...], preferred_element_type=jnp.float32)
    zw_ref[...] = zw
    a_s = jnp.sum(zw * aw_src_ref[...][None, :], axis=1)
    a_d = jnp.sum(zw * aw_dst_ref[...][None, :], axis=1)
    as_ref[...] = a_s
    ad_ref[...] = a_d
    cb = jnp.max(a_s) + jnp.max(a_d)
    cb = jnp.where(cb > 0, cb, 0.2 * cb)
    c_ref[...] = jnp.full((128,), cb, jnp.float32)


def _tc3_body(x1_ref, zw_ref, accp2_ref, denp_ref, as_ref, ad_ref, gatb_ref,
              gnw_ref, gnb_ref, gnms_ref, out_ref):
    a_s = as_ref[...]
    a_d = ad_ref[...]
    cb = jnp.max(a_s) + jnp.max(a_d)
    cb = jnp.where(cb > 0, cb, 0.2 * cb)
    es = a_s + a_d
    es = jnp.where(es > 0, es, 0.2 * es)
    ee_self = jnp.exp(es - cb)
    den = denp_ref[0] + denp_ref[1] + ee_self
    zw = zw_ref[...]
    acc = accp2_ref[0] + accp2_ref[1] + ee_self[:, None] * zw
    h2 = acc / (den + 1e-16)[:, None] + gatb_ref[...]
    mean = jnp.mean(h2, axis=0, keepdims=True)
    cen = h2 - mean * gnms_ref[...]
    var = jnp.mean(cen * cen, axis=0, keepdims=True)
    h2 = gnw_ref[...] * cen / jnp.sqrt(var + 1e-5) + gnb_ref[...]
    h2 = jnp.where(h2 > 0, h2, 0.01 * h2)
    out_ref[...] = x1_ref[...] + h2


def _tc_call(body, out_shapes, *args):
    return pl.pallas_call(
        body,
        out_shape=out_shapes,
    )(*args)


def kernel(x, edges, W1, b1, gn_w, gn_b, gn_ms, gat_W, att_src, att_dst,
           gat_b):
    pad = EPAD - E
    pad_i = jnp.arange(pad, dtype=jnp.int32)
    src_p = jnp.concatenate([edges[0], pad_i % 256])
    dst_p = jnp.concatenate([edges[1], N + (pad_i % (NTRASH - N))])
    src3 = src_p.reshape(NW, NJ, WIN)
    dst3 = dst_p.reshape(NW, NJ, WIN)

    degp = _sc_deg(dst3)[:, :N]
    y = _tc_call(_tc1_body, jax.ShapeDtypeStruct((N, D), jnp.float32),
                 x, W1, degp)
    accp = _sc_gcn(y, src3, dst3)[:, :N, :]
    x1, zw, a_s, a_d, cvec = _tc_call(
        _tc2_body,
        [
            jax.ShapeDtypeStruct((N, D), jnp.float32),
            jax.ShapeDtypeStruct((N, D), jnp.float32),
            jax.ShapeDtypeStruct((N,), jnp.float32),
            jax.ShapeDtypeStruct((N,), jnp.float32),
            jax.ShapeDtypeStruct((128,), jnp.float32),
        ],
        x, y, accp, degp, b1, gn_w, gn_b, gn_ms, gat_W, att_src, att_dst)
    tabs = jnp.pad(a_s, (0, NTRASH - N))
    tabd = jnp.pad(a_d, (0, NTRASH - N))
    ee3, denp = _sc_gat0(src3, dst3, tabs, tabd, cvec)
    denp = denp[:, :N]
    accp2 = _sc_gat(zw, src3, dst3, ee3)[:, :N, :]
    out = _tc_call(_tc3_body, jax.ShapeDtypeStruct((N, D), jnp.float32),
                   x1, zw, accp2, denp, a_s, a_d, gat_b, gn_w, gn_b, gn_ms)
    return out


# speedup vs baseline: 1.1436x; 1.1436x over previous
import functools

import jax
import jax.numpy as jnp
from jax import lax
from jax.experimental import pallas as pl
from jax.experimental.pallas import tpu as pltpu
from jax.experimental.pallas import tpu_sc as plsc

N = 10000
E = 320000
D = 128

NC = 2
NS = 16
NW = NC * NS
WIN = 128
NJ = 80
NG = NJ // 2
EPW = NJ * WIN
EPAD = NW * EPW
NTRASH = 10112
RPS = NTRASH // NS


def _mesh():
    return plsc.VectorSubcoreMesh(core_axis_name="c", subcore_axis_name="s")


_SC_PARAMS = pltpu.CompilerParams(needs_layout_passes=False)


@functools.partial(
    pl.kernel,
    mesh=_mesh(),
    compiler_params=_SC_PARAMS,
    out_type=jax.ShapeDtypeStruct((NC, NTRASH), jnp.float32),
    scratch_types=[
        pltpu.VMEM((NJ, WIN), jnp.int32),
        pltpu.VMEM((WIN,), jnp.float32),
        pltpu.VMEM((NTRASH,), jnp.float32),
        pltpu.VMEM_SHARED((NTRASH,), jnp.float32),
    ],
)
def _sc_deg(dst3, out, dst_buf, ones_v, zbuf, deg_sh):
    cid = lax.axis_index("c")
    sid = lax.axis_index("s")
    wid = cid * NS + sid

    def fill_ones(i, _):
        ones_v[pl.ds(i * 16, 16)] = jnp.ones((16,), jnp.float32)
        return 0

    lax.fori_loop(0, WIN // 16, fill_ones, 0)

    def fill_zero(i, _):
        zbuf[pl.ds(i * 16, 16)] = jnp.zeros((16,), jnp.float32)
        return 0

    lax.fori_loop(0, NTRASH // 16, fill_zero, 0)

    @pl.when(sid == 0)
    def _():
        pltpu.sync_copy(zbuf, deg_sh)

    plsc.subcore_barrier()
    pltpu.sync_copy(dst3.at[wid], dst_buf)

    def body(j, _):
        pltpu.sync_copy(ones_v, deg_sh.at[dst_buf.at[j]], add=True)
        return 0

    lax.fori_loop(0, NJ, body, 0)
    plsc.subcore_barrier()

    @pl.when(sid == 0)
    def _():
        pltpu.sync_copy(deg_sh, out.at[cid])


@functools.partial(
    pl.kernel,
    mesh=_mesh(),
    compiler_params=_SC_PARAMS,
    out_type=jax.ShapeDtypeStruct((NC, NTRASH, D), jnp.float32),
    scratch_types=[
        pltpu.VMEM((WIN,), jnp.int32),
        pltpu.VMEM((WIN,), jnp.int32),
        pltpu.VMEM((WIN,), jnp.int32),
        pltpu.VMEM((WIN,), jnp.int32),
        pltpu.VMEM((WIN, D), jnp.float32),
        pltpu.VMEM((WIN, D), jnp.float32),
        pltpu.SemaphoreType.DMA,
        pltpu.SemaphoreType.DMA,
        pltpu.SemaphoreType.DMA,
        pltpu.SemaphoreType.DMA,
        pltpu.VMEM_SHARED((NTRASH, D), jnp.float32),
    ],
)
def _sc_gcn(y_hbm, src3, dst3, out, src_w0, dst_w0, src_w1, dst_w1,
            rows0, rows1, gsem0, gsem1, ssem0, ssem1, acc_sh):
    cid = lax.axis_index("c")
    sid = lax.axis_index("s")
    wid = cid * NS + sid

    def fill_zero(i, _):
        for k in range(D // 16):
            rows0[i, pl.ds(k * 16, 16)] = jnp.zeros((16,), jnp.float32)
        return 0

    lax.fori_loop(0, WIN, fill_zero, 0)
    for k in range(RPS // WIN):
        pltpu.sync_copy(rows0, acc_sh.at[pl.ds(sid * RPS + k * WIN, WIN)])
    pltpu.sync_copy(rows0.at[pl.ds(0, RPS % WIN)],
                    acc_sh.at[pl.ds(sid * RPS + RPS - RPS % WIN, RPS % WIN)])
    plsc.subcore_barrier()

    pltpu.sync_copy(src3.at[wid, 0], src_w0)
    pltpu.sync_copy(dst3.at[wid, 0], dst_w0)
    g0 = pltpu.async_copy(y_hbm.at[src_w0], rows0, gsem0)

    def pair(g, _):
        j1 = 2 * g + 1

        @pl.when(g > 0)
        def _():
            pltpu.make_async_copy(rows1, acc_sh.at[dst_w1], ssem1).wait()

        pltpu.sync_copy(src3.at[wid, j1], src_w1)
        pltpu.sync_copy(dst3.at[wid, j1], dst_w1)
        pltpu.async_copy(y_hbm.at[src_w1], rows1, gsem1)
        pltpu.make_async_copy(y_hbm.at[src_w0], rows0, gsem0).wait()
        pltpu.async_copy(rows0, acc_sh.at[dst_w0], ssem0, add=True)
        pltpu.make_async_copy(rows0, acc_sh.at[dst_w0], ssem0).wait()

        @pl.when(g < NG - 1)
        def _():
            pltpu.sync_copy(src3.at[wid, 2 * g + 2], src_w0)
            pltpu.sync_copy(dst3.at[wid, 2 * g + 2], dst_w0)
            pltpu.async_copy(y_hbm.at[src_w0], rows0, gsem0)

        pltpu.make_async_copy(y_hbm.at[src_w1], rows1, gsem1).wait()
        pltpu.async_copy(rows1, acc_sh.at[dst_w1], ssem1, add=True)
        return 0

    lax.fori_loop(0, NG, pair, 0)
    pltpu.make_async_copy(rows1, acc_sh.at[dst_w1], ssem1).wait()
    plsc.subcore_barrier()
    pltpu.sync_copy(acc_sh.at[pl.ds(sid * RPS, RPS)],
                    out.at[cid, pl.ds(sid * RPS, RPS)])


@functools.partial(
    pl.kernel,
    mesh=_mesh(),
    compiler_params=_SC_PARAMS,
    out_type=[
        jax.ShapeDtypeStruct((NW, NJ, WIN), jnp.float32),
        jax.ShapeDtypeStruct((NC, NTRASH), jnp.float32),
    ],
    scratch_types=[
        pltpu.VMEM((NJ, WIN), jnp.int32),
        pltpu.VMEM((NJ, WIN), jnp.int32),
        pltpu.VMEM((NJ, WIN), jnp.float32),
        pltpu.VMEM((NTRASH,), jnp.float32),
        pltpu.VMEM((NTRASH,), jnp.float32),
        pltpu.VMEM((128,), jnp.float32),
        pltpu.VMEM_SHARED((NTRASH,), jnp.float32),
    ],
)
def _sc_gat0(src3, dst3, tabs_hbm, tabd_hbm, c_hbm, out_ee, out_den,
             src_buf, dst_buf, ee_buf, asrc_v, adst_v, c_v, den_sh):
    cid = lax.axis_index("c")
    sid = lax.axis_index("s")
    wid = cid * NS + sid

    def fill_zero(i, _):
        asrc_v[pl.ds(i * 16, 16)] = jnp.zeros((16,), jnp.float32)
        return 0

    lax.fori_loop(0, NTRASH // 16, fill_zero, 0)

    @pl.when(sid == 0)
    def _():
        pltpu.sync_copy(asrc_v, den_sh)

    plsc.subcore_barrier()

    pltpu.sync_copy(tabs_hbm, asrc_v)
    pltpu.sync_copy(tabd_hbm, adst_v)
    pltpu.sync_copy(c_hbm, c_v)
    pltpu.sync_copy(src3.at[wid], src_buf)
    pltpu.sync_copy(dst3.at[wid], dst_buf)
    c16 = c_v[pl.ds(0, 16)]

    def body(j, _):
        for k in range(WIN // 16):
            sl = pl.ds(k * 16, 16)
            si = src_buf[j, sl]
            di = dst_buf[j, sl]
            t = plsc.load_gather(asrc_v, [si]) + plsc.load_gather(adst_v, [di])
            t = jnp.where(t > 0, t, 0.2 * t)
            ee_buf[j, sl] = jnp.exp(t - c16)
        pltpu.sync_copy(ee_buf.at[j], den_sh.at[dst_buf.at[j]], add=True)
        return 0

    lax.fori_loop(0, NJ, body, 0)
    pltpu.sync_copy(ee_buf, out_ee.at[wid])
    plsc.subcore_barrier()

    @pl.when(sid == 0)
    def _():
        pltpu.sync_copy(den_sh, out_den.at[cid])


@functools.partial(
    pl.kernel,
    mesh=_mesh(),
    compiler_params=_SC_PARAMS,
    out_type=jax.ShapeDtypeStruct((NC, NTRASH, D), jnp.float32),
    scratch_types=[
        pltpu.VMEM((WIN,), jnp.int32),
        pltpu.VMEM((WIN,), jnp.int32),
        pltpu.VMEM((WIN,), jnp.int32),
        pltpu.VMEM((WIN,), jnp.int32),
        pltpu.VMEM((WIN, D), jnp.float32),
        pltpu.VMEM((WIN, D), jnp.float32),
        pltpu.VMEM((WIN,), jnp.float32),
        pltpu.VMEM((WIN,), jnp.float32),
        pltpu.SemaphoreType.DMA,
        pltpu.SemaphoreType.DMA,
        pltpu.SemaphoreType.DMA,
        pltpu.SemaphoreType.DMA,
        pltpu.VMEM_SHARED((NTRASH, D), jnp.float32),
    ],
)
def _sc_gat(zw_hbm, src3, dst3, ee3, out, src_w0, dst_w0, src_w1, dst_w1,
            rows0, rows1, ee_w0, ee_w1, gsem0, gsem1, ssem0, ssem1, acc_sh):
    cid = lax.axis_index("c")
    sid = lax.axis_index("s")
    wid = cid * NS + sid

    def fill_zero(i, _):
        for k in range(D // 16):
            rows0[i, pl.ds(k * 16, 16)] = jnp.zeros((16,), jnp.float32)
        return 0

    lax.fori_loop(0, WIN, fill_zero, 0)
    for k in range(RPS // WIN):
        pltpu.sync_copy(rows0, acc_sh.at[pl.ds(sid * RPS + k * WIN, WIN)])
    pltpu.sync_copy(rows0.at[pl.ds(0, RPS % WIN)],
                    acc_sh.at[pl.ds(sid * RPS + RPS - RPS % WIN, RPS % WIN)])
    plsc.subcore_barrier()

    def scale(rows, ee_w):
        def scale_grp(gi, _):
            ee16 = ee_w[pl.ds(gi * 16, 16)]
            for u in range(16):
                rr = gi * 16 + u
                s = jnp.take_along_axis(
                    ee16, jnp.full((16,), u, jnp.int32), axis=0)
                for k in range(D // 16):
                    sl = pl.ds(k * 16, 16)
                    rows[rr, sl] = rows[rr, sl] * s
            return 0

        lax.fori_loop(0, WIN // 16, scale_grp, 0)

    pltpu.sync_copy(src3.at[wid, 0], src_w0)
    pltpu.sync_copy(dst3.at[wid, 0], dst_w0)
    pltpu.sync_copy(ee3.at[wid, 0], ee_w0)
    pltpu.async_copy(zw_hbm.at[src_w0], rows0, gsem0)

    def pair(g, _):
        j0 = 2 * g
        j1 = j0 + 1

        @pl.when(g > 0)
        def _():
            pltpu.make_async_copy(rows1, acc_sh.at[dst_w1], ssem1).wait()

        pltpu.sync_copy(src3.at[wid, j1], src_w1)
        pltpu.sync_copy(dst3.at[wid, j1], dst_w1)
        pltpu.sync_copy(ee3.at[wid, j1], ee_w1)
        pltpu.async_copy(zw_hbm.at[src_w1], rows1, gsem1)
        pltpu.make_async_copy(zw_hbm.at[src_w0], rows0, gsem0).wait()
        scale(rows0, ee_w0)
        pltpu.async_copy(rows0, acc_sh.at[dst_w0], ssem0, add=True)
        pltpu.make_async_copy(zw_hbm.at[src_w1], rows1, gsem1).wait()
        scale(rows1, ee_w1)
        pltpu.make_async_copy(rows0, acc_sh.at[dst_w0], ssem0).wait()

        @pl.when(g < NG - 1)
        def _():
            pltpu.sync_copy(src3.at[wid, j0 + 2], src_w0)
            pltpu.sync_copy(dst3.at[wid, j0 + 2], dst_w0)
            pltpu.sync_copy(ee3.at[wid, j0 + 2], ee_w0)
            pltpu.async_copy(zw_hbm.at[src_w0], rows0, gsem0)

        pltpu.async_copy(rows1, acc_sh.at[dst_w1], ssem1, add=True)
        return 0

    lax.fori_loop(0, NG, pair, 0)
    pltpu.make_async_copy(rows1, acc_sh.at[dst_w1], ssem1).wait()
    plsc.subcore_barrier()
    pltpu.sync_copy(acc_sh.at[pl.ds(sid * RPS, RPS)],
                    out.at[cid, pl.ds(sid * RPS, RPS)])


def _tc1_body(x_ref, w_ref, degp_ref, y_ref):
    deg = degp_ref[0] + degp_ref[1] + 1.0
    dinv = 1.0 / jnp.sqrt(deg)
    xw = jnp.dot(x_ref[...], w_ref[...], preferred_element_type=jnp.float32)
    y_ref[...] = dinv[:, None] * xw


def _tc2_body(x_ref, y_ref, accp_ref, degp_ref, b1_ref, gnw_ref, gnb_ref,
              gnms_ref, gatw_ref, aw_src_ref, aw_dst_ref,
              x1_ref, zw_ref, as_ref, ad_ref, c_ref):
    deg = degp_ref[0] + degp_ref[1] + 1.0
    dinv = 1.0 / jnp.sqrt(deg)
    acc = accp_ref[0] + accp_ref[1]
    h = dinv[:, None] * (acc + y_ref[...]) + b1_ref[...]
    mean = jnp.mean(h, axis=0, keepdims=True)
    cen = h - mean * gnms_ref[...]
    var = jnp.mean(cen * cen, axis=0, keepdims=True)
    h = gnw_ref[...] * cen / jnp.sqrt(var + 1e-5) + gnb_ref[...]
    h = jnp.where(h > 0, h, 0.01 * h)
    x1 = x_ref[...] + h
    x1_ref[...] = x1
    zw = jnp.dot(x1, gatw_ref[...], preferred_element_type=jnp.float32)
    zw_ref[...] = zw
    a_s = jnp.sum(zw * aw_src_ref[...][None, :], axis=1)
    a_d = jnp.sum(zw * aw_dst_ref[...][None, :], axis=1)
    as_ref[...] = a_s
    ad_ref[...] = a_d
    cb = jnp.max(a_s) + jnp.max(a_d)
    cb = jnp.where(cb > 0, cb, 0.2 * cb)
    c_ref[...] = jnp.full((128,), cb, jnp.float32)


def _tc3_body(x1_ref, zw_ref, accp2_ref, denp_ref, as_ref, ad_ref, gatb_ref,
              gnw_ref, gnb_ref, gnms_ref, out_ref):
    a_s = as_ref[...]
    a_d = ad_ref[...]
    cb = jnp.max(a_s) + jnp.max(a_d)
    cb = jnp.where(cb > 0, cb, 0.2 * cb)
    es = a_s + a_d
    es = jnp.where(es > 0, es, 0.2 * es)
    ee_self = jnp.exp(es - cb)
    den = denp_ref[0] + denp_ref[1] + ee_self
    zw = zw_ref[...]
    acc = accp2_ref[0] + accp2_ref[1] + ee_self[:, None] * zw
    h2 = acc / (den + 1e-16)[:, None] + gatb_ref[...]
    mean = jnp.mean(h2, axis=0, keepdims=True)
    cen = h2 - mean * gnms_ref[...]
    var = jnp.mean(cen * cen, axis=0, keepdims=True)
    h2 = gnw_ref[...] * cen / jnp.sqrt(var + 1e-5) + gnb_ref[...]
    h2 = jnp.where(h2 > 0, h2, 0.01 * h2)
    out_ref[...] = x1_ref[...] + h2


def _tc_call(body, out_shapes, *args):
    return pl.pallas_call(
        body,
        out_shape=out_shapes,
    )(*args)


def kernel(x, edges, W1, b1, gn_w, gn_b, gn_ms, gat_W, att_src, att_dst,
           gat_b):
    pad = EPAD - E
    pad_i = jnp.arange(pad, dtype=jnp.int32)
    src_p = jnp.concatenate([edges[0], pad_i % 256])
    dst_p = jnp.concatenate([edges[1], N + (pad_i % (NTRASH - N))])
    src3 = src_p.reshape(NW, NJ, WIN)
    dst3 = dst_p.reshape(NW, NJ, WIN)

    degp = _sc_deg(dst3)[:, :N]
    y = _tc_call(_tc1_body, jax.ShapeDtypeStruct((N, D), jnp.float32),
                 x, W1, degp)
    accp = _sc_gcn(y, src3, dst3)[:, :N, :]
    x1, zw, a_s, a_d, cvec = _tc_call(
        _tc2_body,
        [
            jax.ShapeDtypeStruct((N, D), jnp.float32),
            jax.ShapeDtypeStruct((N, D), jnp.float32),
            jax.ShapeDtypeStruct((N,), jnp.float32),
            jax.ShapeDtypeStruct((N,), jnp.float32),
            jax.ShapeDtypeStruct((128,), jnp.float32),
        ],
        x, y, accp, degp, b1, gn_w, gn_b, gn_ms, gat_W, att_src, att_dst)
    tabs = jnp.pad(a_s, (0, NTRASH - N))
    tabd = jnp.pad(a_d, (0, NTRASH - N))
    ee3, denp = _sc_gat0(src3, dst3, tabs, tabd, cvec)
    denp = denp[:, :N]
    accp2 = _sc_gat(zw, src3, dst3, ee3)[:, :N, :]
    out = _tc_call(_tc3_body, jax.ShapeDtypeStruct((N, D), jnp.float32),
                   x1, zw, accp2, denp, a_s, a_d, gat_b, gn_w, gn_b, gn_ms)
    return out


# speedup vs baseline: 1.2928x; 1.1305x over previous
import functools

import jax
import jax.numpy as jnp
from jax import lax
from jax.experimental import pallas as pl
from jax.experimental.pallas import tpu as pltpu
from jax.experimental.pallas import tpu_sc as plsc

N = 10000
E = 320000
D = 128

NC = 2
NS = 16
NW = NC * NS
WIN = 128
NJ = 80
NG = NJ // 2
EPW = NJ * WIN
EPAD = NW * EPW
NTRASH = 10112
RPS = NTRASH // NS


def _mesh():
    return plsc.VectorSubcoreMesh(core_axis_name="c", subcore_axis_name="s")


_SC_PARAMS = pltpu.CompilerParams(needs_layout_passes=False)


@functools.partial(
    pl.kernel,
    mesh=_mesh(),
    compiler_params=_SC_PARAMS,
    out_type=jax.ShapeDtypeStruct((NC, NTRASH), jnp.float32),
    scratch_types=[
        pltpu.VMEM((NJ, WIN), jnp.int32),
        pltpu.VMEM((WIN,), jnp.float32),
        pltpu.VMEM((NTRASH,), jnp.float32),
        pltpu.VMEM_SHARED((NTRASH,), jnp.float32),
    ],
)
def _sc_deg(dst3, out, dst_buf, ones_v, zbuf, deg_sh):
    cid = lax.axis_index("c")
    sid = lax.axis_index("s")
    wid = cid * NS + sid

    def fill_ones(i, _):
        ones_v[pl.ds(i * 16, 16)] = jnp.ones((16,), jnp.float32)
        return 0

    lax.fori_loop(0, WIN // 16, fill_ones, 0)

    def fill_zero(i, _):
        zbuf[pl.ds(i * 16, 16)] = jnp.zeros((16,), jnp.float32)
        return 0

    lax.fori_loop(0, NTRASH // 16, fill_zero, 0)

    @pl.when(sid == 0)
    def _():
        pltpu.sync_copy(zbuf, deg_sh)

    plsc.subcore_barrier()
    pltpu.sync_copy(dst3.at[wid], dst_buf)

    def body(j, _):
        pltpu.sync_copy(ones_v, deg_sh.at[dst_buf.at[j]], add=True)
        return 0

    lax.fori_loop(0, NJ, body, 0)
    plsc.subcore_barrier()

    @pl.when(sid == 0)
    def _():
        pltpu.sync_copy(deg_sh, out.at[cid])


@functools.partial(
    pl.kernel,
    mesh=_mesh(),
    compiler_params=_SC_PARAMS,
    out_type=jax.ShapeDtypeStruct((NC, NTRASH, D), jnp.float32),
    scratch_types=[
        pltpu.VMEM((2, WIN), jnp.int32),
        pltpu.VMEM((2, WIN), jnp.int32),
        pltpu.VMEM((WIN, D), jnp.float32),
        pltpu.VMEM((WIN, D), jnp.float32),
        pltpu.SemaphoreType.DMA,
        pltpu.SemaphoreType.DMA,
        pltpu.SemaphoreType.DMA,
        pltpu.SemaphoreType.DMA,
        pltpu.VMEM_SHARED((NTRASH, D), jnp.float32),
    ],
)
def _sc_gcn(y_hbm, meta, out, mb0, mb1,
            rows0, rows1, gsem0, gsem1, ssem0, ssem1, acc_sh):
    cid = lax.axis_index("c")
    sid = lax.axis_index("s")
    wid = cid * NS + sid

    def fill_zero(i, _):
        for k in range(D // 16):
            rows0[i, pl.ds(k * 16, 16)] = jnp.zeros((16,), jnp.float32)
        return 0

    lax.fori_loop(0, WIN, fill_zero, 0)
    for k in range(RPS // WIN):
        pltpu.sync_copy(rows0, acc_sh.at[pl.ds(sid * RPS + k * WIN, WIN)])
    pltpu.sync_copy(rows0.at[pl.ds(0, RPS % WIN)],
                    acc_sh.at[pl.ds(sid * RPS + RPS - RPS % WIN, RPS % WIN)])
    plsc.subcore_barrier()

    pltpu.sync_copy(meta.at[wid, 0], mb0)
    pltpu.async_copy(y_hbm.at[mb0.at[0]], rows0, gsem0)

    def pair(g, _):
        j1 = 2 * g + 1

        @pl.when(g > 0)
        def _():
            pltpu.make_async_copy(rows1, acc_sh.at[mb1.at[1]], ssem1).wait()

        pltpu.sync_copy(meta.at[wid, j1], mb1)
        pltpu.async_copy(y_hbm.at[mb1.at[0]], rows1, gsem1)
        pltpu.make_async_copy(y_hbm.at[mb0.at[0]], rows0, gsem0).wait()
        pltpu.async_copy(rows0, acc_sh.at[mb0.at[1]], ssem0, add=True)
        pltpu.make_async_copy(rows0, acc_sh.at[mb0.at[1]], ssem0).wait()

        @pl.when(g < NG - 1)
        def _():
            pltpu.sync_copy(meta.at[wid, 2 * g + 2], mb0)
            pltpu.async_copy(y_hbm.at[mb0.at[0]], rows0, gsem0)

        pltpu.make_async_copy(y_hbm.at[mb1.at[0]], rows1, gsem1).wait()
        pltpu.async_copy(rows1, acc_sh.at[mb1.at[1]], ssem1, add=True)
        return 0

    lax.fori_loop(0, NG, pair, 0)
    pltpu.make_async_copy(rows1, acc_sh.at[mb1.at[1]], ssem1).wait()
    plsc.subcore_barrier()
    pltpu.sync_copy(acc_sh.at[pl.ds(sid * RPS, RPS)],
                    out.at[cid, pl.ds(sid * RPS, RPS)])


@functools.partial(
    pl.kernel,
    mesh=_mesh(),
    compiler_params=_SC_PARAMS,
    out_type=[
        jax.ShapeDtypeStruct((NW, NJ, WIN), jnp.float32),
        jax.ShapeDtypeStruct((NC, NTRASH), jnp.float32),
    ],
    scratch_types=[
        pltpu.VMEM((NJ, WIN), jnp.int32),
        pltpu.VMEM((NJ, WIN), jnp.int32),
        pltpu.VMEM((NJ, WIN), jnp.float32),
        pltpu.VMEM((NTRASH,), jnp.float32),
        pltpu.VMEM((NTRASH,), jnp.float32),
        pltpu.VMEM((128,), jnp.float32),
        pltpu.VMEM_SHARED((NTRASH,), jnp.float32),
    ],
)
def _sc_gat0(src3, dst3, tabs_hbm, tabd_hbm, c_hbm, out_ee, out_den,
             src_buf, dst_buf, ee_buf, asrc_v, adst_v, c_v, den_sh):
    cid = lax.axis_index("c")
    sid = lax.axis_index("s")
    wid = cid * NS + sid

    def fill_zero(i, _):
        asrc_v[pl.ds(i * 16, 16)] = jnp.zeros((16,), jnp.float32)
        return 0

    lax.fori_loop(0, NTRASH // 16, fill_zero, 0)

    @pl.when(sid == 0)
    def _():
        pltpu.sync_copy(asrc_v, den_sh)

    plsc.subcore_barrier()

    pltpu.sync_copy(tabs_hbm, asrc_v)
    pltpu.sync_copy(tabd_hbm, adst_v)
    pltpu.sync_copy(c_hbm, c_v)
    pltpu.sync_copy(src3.at[wid], src_buf)
    pltpu.sync_copy(dst3.at[wid], dst_buf)
    c16 = c_v[pl.ds(0, 16)]

    def body(j, _):
        for k in range(WIN // 16):
            sl = pl.ds(k * 16, 16)
            si = src_buf[j, sl]
            di = dst_buf[j, sl]
            t = plsc.load_gather(asrc_v, [si]) + plsc.load_gather(adst_v, [di])
            t = jnp.where(t > 0, t, 0.2 * t)
            ee_buf[j, sl] = jnp.exp(t - c16)
        pltpu.sync_copy(ee_buf.at[j], den_sh.at[dst_buf.at[j]], add=True)
        return 0

    lax.fori_loop(0, NJ, body, 0)
    pltpu.sync_copy(ee_buf, out_ee.at[wid])
    plsc.subcore_barrier()

    @pl.when(sid == 0)
    def _():
        pltpu.sync_copy(den_sh, out_den.at[cid])


@functools.partial(
    pl.kernel,
    mesh=_mesh(),
    compiler_params=_SC_PARAMS,
    out_type=jax.ShapeDtypeStruct((NC, NTRASH, D), jnp.float32),
    scratch_types=[
        pltpu.VMEM((2, WIN), jnp.int32),
        pltpu.VMEM((2, WIN), jnp.int32),
        pltpu.VMEM((WIN, D), jnp.float32),
        pltpu.VMEM((WIN, D), jnp.float32),
        pltpu.VMEM((WIN,), jnp.float32),
        pltpu.VMEM((WIN,), jnp.float32),
        pltpu.SemaphoreType.DMA,
        pltpu.SemaphoreType.DMA,
        pltpu.SemaphoreType.DMA,
        pltpu.SemaphoreType.DMA,
        pltpu.VMEM_SHARED((NTRASH, D), jnp.float32),
    ],
)
def _sc_gat(zw_hbm, meta, ee3, out, mb0, mb1,
            rows0, rows1, ee_w0, ee_w1, gsem0, gsem1, ssem0, ssem1, acc_sh):
    cid = lax.axis_index("c")
    sid = lax.axis_index("s")
    wid = cid * NS + sid

    def fill_zero(i, _):
        for k in range(D // 16):
            rows0[i, pl.ds(k * 16, 16)] = jnp.zeros((16,), jnp.float32)
        return 0

    lax.fori_loop(0, WIN, fill_zero, 0)
    for k in range(RPS // WIN):
        pltpu.sync_copy(rows0, acc_sh.at[pl.ds(sid * RPS + k * WIN, WIN)])
    pltpu.sync_copy(rows0.at[pl.ds(0, RPS % WIN)],
                    acc_sh.at[pl.ds(sid * RPS + RPS - RPS % WIN, RPS % WIN)])
    plsc.subcore_barrier()

    def scale(rows, ee_w):
        def scale_grp(gi, _):
            ee16 = ee_w[pl.ds(gi * 16, 16)]
            for u in range(16):
                rr = gi * 16 + u
                s = jnp.take_along_axis(
                    ee16, jnp.full((16,), u, jnp.int32), axis=0)
                for k in range(D // 16):
                    sl = pl.ds(k * 16, 16)
                    rows[rr, sl] = rows[rr, sl] * s
            return 0

        lax.fori_loop(0, WIN // 16, scale_grp, 0)

    pltpu.sync_copy(meta.at[wid, 0], mb0)
    pltpu.sync_copy(ee3.at[wid, 0], ee_w0)
    pltpu.async_copy(zw_hbm.at[mb0.at[0]], rows0, gsem0)

    def pair(g, _):
        j0 = 2 * g
        j1 = j0 + 1

        @pl.when(g > 0)
        def _():
            pltpu.make_async_copy(rows1, acc_sh.at[mb1.at[1]], ssem1).wait()

        pltpu.sync_copy(meta.at[wid, j1], mb1)
        pltpu.sync_copy(ee3.at[wid, j1], ee_w1)
        pltpu.async_copy(zw_hbm.at[mb1.at[0]], rows1, gsem1)
        pltpu.make_async_copy(zw_hbm.at[mb0.at[0]], rows0, gsem0).wait()
        scale(rows0, ee_w0)
        pltpu.async_copy(rows0, acc_sh.at[mb0.at[1]], ssem0, add=True)
        pltpu.make_async_copy(zw_hbm.at[mb1.at[0]], rows1, gsem1).wait()
        scale(rows1, ee_w1)
        pltpu.make_async_copy(rows0, acc_sh.at[mb0.at[1]], ssem0).wait()

        @pl.when(g < NG - 1)
        def _():
            pltpu.sync_copy(meta.at[wid, j0 + 2], mb0)
            pltpu.sync_copy(ee3.at[wid, j0 + 2], ee_w0)
            pltpu.async_copy(zw_hbm.at[mb0.at[0]], rows0, gsem0)

        pltpu.async_copy(rows1, acc_sh.at[mb1.at[1]], ssem1, add=True)
        return 0

    lax.fori_loop(0, NG, pair, 0)
    pltpu.make_async_copy(rows1, acc_sh.at[mb1.at[1]], ssem1).wait()
    plsc.subcore_barrier()
    pltpu.sync_copy(acc_sh.at[pl.ds(sid * RPS, RPS)],
                    out.at[cid, pl.ds(sid * RPS, RPS)])


def _tc1_body(x_ref, w_ref, degp_ref, y_ref):
    deg = degp_ref[0] + degp_ref[1] + 1.0
    dinv = 1.0 / jnp.sqrt(deg)
    xw = jnp.dot(x_ref[...], w_ref[...], preferred_element_type=jnp.float32)
    y_ref[...] = dinv[:, None] * xw


def _tc2_body(x_ref, y_ref, accp_ref, degp_ref, b1_ref, gnw_ref, gnb_ref,
              gnms_ref, gatw_ref, aw_src_ref, aw_dst_ref,
              x1_ref, zw_ref, as_ref, ad_ref, c_ref):
    deg = degp_ref[0] + degp_ref[1] + 1.0
    dinv = 1.0 / jnp.sqrt(deg)
    acc = accp_ref[0] + accp_ref[1]
    h = dinv[:, None] * (acc + y_ref[...]) + b1_ref[...]
    mean = jnp.mean(h, axis=0, keepdims=True)
    cen = h - mean * gnms_ref[...]
    var = jnp.mean(cen * cen, axis=0, keepdims=True)
    h = gnw_ref[...] * cen / jnp.sqrt(var + 1e-5) + gnb_ref[...]
    h = jnp.where(h > 0, h, 0.01 * h)
    x1 = x_ref[...] + h
    x1_ref[...] = x1
    zw = jnp.dot(x1, gatw_ref[...], preferred_element_type=jnp.float32)
    zw_ref[...] = zw
    a_s = jnp.sum(zw * aw_src_ref[...][None, :], axis=1)
    a_d = jnp.sum(zw * aw_dst_ref[...][None, :], axis=1)
    as_ref[...] = a_s
    ad_ref[...] = a_d
    cb = jnp.max(a_s) + jnp.max(a_d)
    cb = jnp.where(cb > 0, cb, 0.2 * cb)
    c_ref[...] = jnp.full((128,), cb, jnp.float32)


def _tc3_body(x1_ref, zw_ref, accp2_ref, denp_ref, as_ref, ad_ref, gatb_ref,
              gnw_ref, gnb_ref, gnms_ref, out_ref):
    a_s = as_ref[...]
    a_d = ad_ref[...]
    cb = jnp.max(a_s) + jnp.max(a_d)
    cb = jnp.where(cb > 0, cb, 0.2 * cb)
    es = a_s + a_d
    es = jnp.where(es > 0, es, 0.2 * es)
    ee_self = jnp.exp(es - cb)
    den = denp_ref[0] + denp_ref[1] + ee_self
    zw = zw_ref[...]
    acc = accp2_ref[0] + accp2_ref[1] + ee_self[:, None] * zw
    h2 = acc / (den + 1e-16)[:, None] + gatb_ref[...]
    mean = jnp.mean(h2, axis=0, keepdims=True)
    cen = h2 - mean * gnms_ref[...]
    var = jnp.mean(cen * cen, axis=0, keepdims=True)
    h2 = gnw_ref[...] * cen / jnp.sqrt(var + 1e-5) + gnb_ref[...]
    h2 = jnp.where(h2 > 0, h2, 0.01 * h2)
    out_ref[...] = x1_ref[...] + h2


def _tc_call(body, out_shapes, *args):
    return pl.pallas_call(
        body,
        out_shape=out_shapes,
    )(*args)


def kernel(x, edges, W1, b1, gn_w, gn_b, gn_ms, gat_W, att_src, att_dst,
           gat_b):
    pad = EPAD - E
    pad_i = jnp.arange(pad, dtype=jnp.int32)
    src_p = jnp.concatenate([edges[0], pad_i % 256])
    dst_p = jnp.concatenate([edges[1], N + (pad_i % (NTRASH - N))])
    src3 = src_p.reshape(NW, NJ, WIN)
    dst3 = dst_p.reshape(NW, NJ, WIN)
    meta = jnp.stack([src3, dst3], axis=2)

    degp = _sc_deg(dst3)[:, :N]
    y = _tc_call(_tc1_body, jax.ShapeDtypeStruct((N, D), jnp.float32),
                 x, W1, degp)
    accp = _sc_gcn(y, meta)[:, :N, :]
    x1, zw, a_s, a_d, cvec = _tc_call(
        _tc2_body,
        [
            jax.ShapeDtypeStruct((N, D), jnp.float32),
            jax.ShapeDtypeStruct((N, D), jnp.float32),
            jax.ShapeDtypeStruct((N,), jnp.float32),
            jax.ShapeDtypeStruct((N,), jnp.float32),
            jax.ShapeDtypeStruct((128,), jnp.float32),
        ],
        x, y, accp, degp, b1, gn_w, gn_b, gn_ms, gat_W, att_src, att_dst)
    tabs = jnp.pad(a_s, (0, NTRASH - N))
    tabd = jnp.pad(a_d, (0, NTRASH - N))
    ee3, denp = _sc_gat0(src3, dst3, tabs, tabd, cvec)
    denp = denp[:, :N]
    accp2 = _sc_gat(zw, meta, ee3)[:, :N, :]
    out = _tc_call(_tc3_body, jax.ShapeDtypeStruct((N, D), jnp.float32),
                   x1, zw, accp2, denp, a_s, a_d, gat_b, gn_w, gn_b, gn_ms)
    return out


# speedup vs baseline: 1.3750x; 1.0636x over previous
import functools

import jax
import jax.numpy as jnp
from jax import lax
from jax.experimental import pallas as pl
from jax.experimental.pallas import tpu as pltpu
from jax.experimental.pallas import tpu_sc as plsc

N = 10000
E = 320000
D = 128

NC = 2
NS = 16
NW = NC * NS
WIN = 128
NJ = 80
NG = NJ // 2
EPW = NJ * WIN
EPAD = NW * EPW
NTRASH = 10112
RPS = NTRASH // NS


def _mesh():
    return plsc.VectorSubcoreMesh(core_axis_name="c", subcore_axis_name="s")


_SC_PARAMS = pltpu.CompilerParams(needs_layout_passes=False)


@functools.partial(
    pl.kernel,
    mesh=_mesh(),
    compiler_params=_SC_PARAMS,
    out_type=jax.ShapeDtypeStruct((NC, NTRASH), jnp.float32),
    scratch_types=[
        pltpu.VMEM((NJ, WIN), jnp.int32),
        pltpu.VMEM((WIN,), jnp.float32),
        pltpu.VMEM((NTRASH,), jnp.float32),
        pltpu.VMEM_SHARED((NTRASH,), jnp.float32),
    ],
)
def _sc_deg(dst3, out, dst_buf, ones_v, zbuf, deg_sh):
    cid = lax.axis_index("c")
    sid = lax.axis_index("s")
    wid = cid * NS + sid

    def fill_ones(i, _):
        ones_v[pl.ds(i * 16, 16)] = jnp.ones((16,), jnp.float32)
        return 0

    lax.fori_loop(0, WIN // 16, fill_ones, 0)

    def fill_zero(i, _):
        zbuf[pl.ds(i * 16, 16)] = jnp.zeros((16,), jnp.float32)
        return 0

    lax.fori_loop(0, NTRASH // 16, fill_zero, 0)

    @pl.when(sid == 0)
    def _():
        pltpu.sync_copy(zbuf, deg_sh)

    plsc.subcore_barrier()
    pltpu.sync_copy(dst3.at[wid], dst_buf)

    def body(j, _):
        pltpu.sync_copy(ones_v, deg_sh.at[dst_buf.at[j]], add=True)
        return 0

    lax.fori_loop(0, NJ, body, 0)
    plsc.subcore_barrier()

    @pl.when(sid == 0)
    def _():
        pltpu.sync_copy(deg_sh, out.at[cid])


@functools.partial(
    pl.kernel,
    mesh=_mesh(),
    compiler_params=_SC_PARAMS,
    out_type=jax.ShapeDtypeStruct((NC, NTRASH, D), jnp.float32),
    scratch_types=[
        pltpu.VMEM((2, WIN), jnp.int32),
        pltpu.VMEM((2, WIN), jnp.int32),
        pltpu.VMEM((WIN, D), jnp.float32),
        pltpu.VMEM((WIN, D), jnp.float32),
        pltpu.SemaphoreType.DMA,
        pltpu.SemaphoreType.DMA,
        pltpu.SemaphoreType.DMA,
        pltpu.SemaphoreType.DMA,
        pltpu.VMEM_SHARED((NTRASH, D), jnp.float32),
    ],
)
def _sc_gcn(y_hbm, meta, out, mb0, mb1,
            rows0, rows1, gsem0, gsem1, ssem0, ssem1, acc_sh):
    cid = lax.axis_index("c")
    sid = lax.axis_index("s")
    wid = cid * NS + sid

    def fill_zero(i, _):
        for k in range(D // 16):
            rows0[i, pl.ds(k * 16, 16)] = jnp.zeros((16,), jnp.float32)
        return 0

    lax.fori_loop(0, WIN, fill_zero, 0)
    for k in range(RPS // WIN):
        pltpu.sync_copy(rows0, acc_sh.at[pl.ds(sid * RPS + k * WIN, WIN)])
    pltpu.sync_copy(rows0.at[pl.ds(0, RPS % WIN)],
                    acc_sh.at[pl.ds(sid * RPS + RPS - RPS % WIN, RPS % WIN)])
    plsc.subcore_barrier()

    pltpu.sync_copy(meta.at[wid, 0], mb0)
    pltpu.async_copy(y_hbm.at[mb0.at[0]], rows0, gsem0)

    def pair(g, _):
        j1 = 2 * g + 1

        @pl.when(g > 0)
        def _():
            pltpu.make_async_copy(rows1, acc_sh.at[mb1.at[1]], ssem1).wait()

        pltpu.sync_copy(meta.at[wid, j1], mb1)
        pltpu.async_copy(y_hbm.at[mb1.at[0]], rows1, gsem1)
        pltpu.make_async_copy(y_hbm.at[mb0.at[0]], rows0, gsem0).wait()
        pltpu.async_copy(rows0, acc_sh.at[mb0.at[1]], ssem0, add=True)
        pltpu.make_async_copy(rows0, acc_sh.at[mb0.at[1]], ssem0).wait()

        @pl.when(g < NG - 1)
        def _():
            pltpu.sync_copy(meta.at[wid, 2 * g + 2], mb0)
            pltpu.async_copy(y_hbm.at[mb0.at[0]], rows0, gsem0)

        pltpu.make_async_copy(y_hbm.at[mb1.at[0]], rows1, gsem1).wait()
        pltpu.async_copy(rows1, acc_sh.at[mb1.at[1]], ssem1, add=True)
        return 0

    lax.fori_loop(0, NG, pair, 0)
    pltpu.make_async_copy(rows1, acc_sh.at[mb1.at[1]], ssem1).wait()
    plsc.subcore_barrier()
    pltpu.sync_copy(acc_sh.at[pl.ds(sid * RPS, RPS)],
                    out.at[cid, pl.ds(sid * RPS, RPS)])


@functools.partial(
    pl.kernel,
    mesh=_mesh(),
    compiler_params=_SC_PARAMS,
    out_type=[
        jax.ShapeDtypeStruct((NW, NJ, WIN), jnp.float32),
        jax.ShapeDtypeStruct((NC, NTRASH), jnp.float32),
    ],
    scratch_types=[
        pltpu.VMEM((NJ, WIN), jnp.int32),
        pltpu.VMEM((NJ, WIN), jnp.int32),
        pltpu.VMEM((NJ, WIN), jnp.float32),
        pltpu.VMEM((NTRASH,), jnp.float32),
        pltpu.VMEM((NTRASH,), jnp.float32),
        pltpu.VMEM((128,), jnp.float32),
        pltpu.VMEM_SHARED((NTRASH,), jnp.float32),
    ],
)
def _sc_gat0(src3, dst3, tabs_hbm, tabd_hbm, c_hbm, out_ee, out_den,
             src_buf, dst_buf, ee_buf, asrc_v, adst_v, c_v, den_sh):
    cid = lax.axis_index("c")
    sid = lax.axis_index("s")
    wid = cid * NS + sid

    def fill_zero(i, _):
        asrc_v[pl.ds(i * 16, 16)] = jnp.zeros((16,), jnp.float32)
        return 0

    lax.fori_loop(0, NTRASH // 16, fill_zero, 0)

    @pl.when(sid == 0)
    def _():
        pltpu.sync_copy(asrc_v, den_sh)

    plsc.subcore_barrier()

    pltpu.sync_copy(tabs_hbm, asrc_v)
    pltpu.sync_copy(tabd_hbm, adst_v)
    pltpu.sync_copy(c_hbm, c_v)
    pltpu.sync_copy(src3.at[wid], src_buf)
    pltpu.sync_copy(dst3.at[wid], dst_buf)
    c16 = c_v[pl.ds(0, 16)]

    def body(j, _):
        for k in range(WIN // 16):
            sl = pl.ds(k * 16, 16)
            si = src_buf[j, sl]
            di = dst_buf[j, sl]
            t = plsc.load_gather(asrc_v, [si]) + plsc.load_gather(adst_v, [di])
            t = jnp.where(t > 0, t, 0.2 * t)
            ee_buf[j, sl] = jnp.exp(t - c16)
        pltpu.sync_copy(ee_buf.at[j], den_sh.at[dst_buf.at[j]], add=True)
        return 0

    lax.fori_loop(0, NJ, body, 0)
    pltpu.sync_copy(ee_buf, out_ee.at[wid])
    plsc.subcore_barrier()

    @pl.when(sid == 0)
    def _():
        pltpu.sync_copy(den_sh, out_den.at[cid])


@functools.partial(
    pl.kernel,
    mesh=_mesh(),
    compiler_params=_SC_PARAMS,
    out_type=jax.ShapeDtypeStruct((NC, NTRASH, D), jnp.float32),
    scratch_types=[
        pltpu.VMEM((3, WIN), jnp.int32),
        pltpu.VMEM((3, WIN), jnp.int32),
        pltpu.VMEM((WIN, D), jnp.float32),
        pltpu.VMEM((WIN, D), jnp.float32),
        pltpu.SemaphoreType.DMA,
        pltpu.SemaphoreType.DMA,
        pltpu.SemaphoreType.DMA,
        pltpu.SemaphoreType.DMA,
        pltpu.VMEM_SHARED((NTRASH, D), jnp.float32),
    ],
)
def _sc_gat(zw_hbm, meta3, out, mb0, mb1,
            rows0, rows1, gsem0, gsem1, ssem0, ssem1, acc_sh):
    cid = lax.axis_index("c")
    sid = lax.axis_index("s")
    wid = cid * NS + sid

    def fill_zero(i, _):
        for k in range(D // 16):
            rows0[i, pl.ds(k * 16, 16)] = jnp.zeros((16,), jnp.float32)
        return 0

    lax.fori_loop(0, WIN, fill_zero, 0)
    for k in range(RPS // WIN):
        pltpu.sync_copy(rows0, acc_sh.at[pl.ds(sid * RPS + k * WIN, WIN)])
    pltpu.sync_copy(rows0.at[pl.ds(0, RPS % WIN)],
                    acc_sh.at[pl.ds(sid * RPS + RPS - RPS % WIN, RPS % WIN)])
    plsc.subcore_barrier()

    def scale(rows, mb):
        def scale_grp(gi, _):
            ee16 = plsc.bitcast(mb[2, pl.ds(gi * 16, 16)], jnp.float32)
            for u in range(16):
                rr = gi * 16 + u
                s = jnp.take_along_axis(
                    ee16, jnp.full((16,), u, jnp.int32), axis=0)
                for k in range(D // 16):
                    sl = pl.ds(k * 16, 16)
                    rows[rr, sl] = rows[rr, sl] * s
            return 0

        lax.fori_loop(0, WIN // 16, scale_grp, 0)

    pltpu.sync_copy(meta3.at[wid, 0], mb0)
    pltpu.async_copy(zw_hbm.at[mb0.at[0]], rows0, gsem0)

    def pair(g, _):
        j0 = 2 * g
        j1 = j0 + 1

        @pl.when(g > 0)
        def _():
            pltpu.make_async_copy(rows1, acc_sh.at[mb1.at[1]], ssem1).wait()

        pltpu.sync_copy(meta3.at[wid, j1], mb1)
        pltpu.async_copy(zw_hbm.at[mb1.at[0]], rows1, gsem1)
        pltpu.make_async_copy(zw_hbm.at[mb0.at[0]], rows0, gsem0).wait()
        scale(rows0, mb0)
        pltpu.async_copy(rows0, acc_sh.at[mb0.at[1]], ssem0, add=True)
        pltpu.make_async_copy(zw_hbm.at[mb1.at[0]], rows1, gsem1).wait()
        scale(rows1, mb1)
        pltpu.make_async_copy(rows0, acc_sh.at[mb0.at[1]], ssem0).wait()

        @pl.when(g < NG - 1)
        def _():
            pltpu.sync_copy(meta3.at[wid, j0 + 2], mb0)
            pltpu.async_copy(zw_hbm.at[mb0.at[0]], rows0, gsem0)

        pltpu.async_copy(rows1, acc_sh.at[mb1.at[1]], ssem1, add=True)
        return 0

    lax.fori_loop(0, NG, pair, 0)
    pltpu.make_async_copy(rows1, acc_sh.at[mb1.at[1]], ssem1).wait()
    plsc.subcore_barrier()
    pltpu.sync_copy(acc_sh.at[pl.ds(sid * RPS, RPS)],
                    out.at[cid, pl.ds(sid * RPS, RPS)])


def _tc1_body(x_ref, w_ref, degp_ref, y_ref):
    deg = degp_ref[0] + degp_ref[1] + 1.0
    dinv = 1.0 / jnp.sqrt(deg)
    xw = jnp.dot(x_ref[...], w_ref[...], preferred_element_type=jnp.float32)
    y_ref[...] = dinv[:, None] * xw


def _tc2_body(x_ref, y_ref, accp_ref, degp_ref, b1_ref, gnw_ref, gnb_ref,
              gnms_ref, gatw_ref, aw_src_ref, aw_dst_ref,
              x1_ref, zw_ref, as_ref, ad_ref, c_ref):
    deg = degp_ref[0] + degp_ref[1] + 1.0
    dinv = 1.0 / jnp.sqrt(deg)
    acc = accp_ref[0] + accp_ref[1]
    h = dinv[:, None] * (acc + y_ref[...]) + b1_ref[...]
    mean = jnp.mean(h, axis=0, keepdims=True)
    cen = h - mean * gnms_ref[...]
    var = jnp.mean(cen * cen, axis=0, keepdims=True)
    h = gnw_ref[...] * cen / jnp.sqrt(var + 1e-5) + gnb_ref[...]
    h = jnp.where(h > 0, h, 0.01 * h)
    x1 = x_ref[...] + h
    x1_ref[...] = x1
    zw = jnp.dot(x1, gatw_ref[...], preferred_element_type=jnp.float32)
    zw_ref[...] = zw
    a_s = jnp.sum(zw * aw_src_ref[...][None, :], axis=1)
    a_d = jnp.sum(zw * aw_dst_ref[...][None, :], axis=1)
    as_ref[...] = a_s
    ad_ref[...] = a_d
    cb = jnp.max(a_s) + jnp.max(a_d)
    cb = jnp.where(cb > 0, cb, 0.2 * cb)
    c_ref[...] = jnp.full((128,), cb, jnp.float32)


def _tc3_body(x1_ref, zw_ref, accp2_ref, denp_ref, as_ref, ad_ref, gatb_ref,
              gnw_ref, gnb_ref, gnms_ref, out_ref):
    a_s = as_ref[...]
    a_d = ad_ref[...]
    cb = jnp.max(a_s) + jnp.max(a_d)
    cb = jnp.where(cb > 0, cb, 0.2 * cb)
    es = a_s + a_d
    es = jnp.where(es > 0, es, 0.2 * es)
    ee_self = jnp.exp(es - cb)
    den = denp_ref[0] + denp_ref[1] + ee_self
    zw = zw_ref[...]
    acc = accp2_ref[0] + accp2_ref[1] + ee_self[:, None] * zw
    h2 = acc / (den + 1e-16)[:, None] + gatb_ref[...]
    mean = jnp.mean(h2, axis=0, keepdims=True)
    cen = h2 - mean * gnms_ref[...]
    var = jnp.mean(cen * cen, axis=0, keepdims=True)
    h2 = gnw_ref[...] * cen / jnp.sqrt(var + 1e-5) + gnb_ref[...]
    h2 = jnp.where(h2 > 0, h2, 0.01 * h2)
    out_ref[...] = x1_ref[...] + h2


def _tc_call(body, out_shapes, *args):
    return pl.pallas_call(
        body,
        out_shape=out_shapes,
    )(*args)


def kernel(x, edges, W1, b1, gn_w, gn_b, gn_ms, gat_W, att_src, att_dst,
           gat_b):
    pad = EPAD - E
    pad_i = jnp.arange(pad, dtype=jnp.int32)
    src_p = jnp.concatenate([edges[0], pad_i % 256])
    dst_p = jnp.concatenate([edges[1], N + (pad_i % (NTRASH - N))])
    src3 = src_p.reshape(NW, NJ, WIN)
    dst3 = dst_p.reshape(NW, NJ, WIN)
    meta = jnp.stack([src3, dst3], axis=2)

    degp = _sc_deg(dst3)[:, :N]
    y = _tc_call(_tc1_body, jax.ShapeDtypeStruct((N, D), jnp.float32),
                 x, W1, degp)
    accp = _sc_gcn(y, meta)[:, :N, :]
    x1, zw, a_s, a_d, cvec = _tc_call(
        _tc2_body,
        [
            jax.ShapeDtypeStruct((N, D), jnp.float32),
            jax.ShapeDtypeStruct((N, D), jnp.float32),
            jax.ShapeDtypeStruct((N,), jnp.float32),
            jax.ShapeDtypeStruct((N,), jnp.float32),
            jax.ShapeDtypeStruct((128,), jnp.float32),
        ],
        x, y, accp, degp, b1, gn_w, gn_b, gn_ms, gat_W, att_src, att_dst)
    tabs = jnp.pad(a_s, (0, NTRASH - N))
    tabd = jnp.pad(a_d, (0, NTRASH - N))
    ee3, denp = _sc_gat0(src3, dst3, tabs, tabd, cvec)
    denp = denp[:, :N]
    meta3 = jnp.concatenate(
        [meta, jax.lax.bitcast_convert_type(ee3, jnp.int32)[:, :, None, :]],
        axis=2)
    accp2 = _sc_gat(zw, meta3)[:, :N, :]
    out = _tc_call(_tc3_body, jax.ShapeDtypeStruct((N, D), jnp.float32),
                   x1, zw, accp2, denp, a_s, a_d, gat_b, gn_w, gn_b, gn_ms)
    return out
